# baseline (speedup 1.0000x reference)
import jax
import jax.numpy as jnp
import numpy as np
from jax.experimental import pallas as pl
from jax.experimental.pallas import tpu as pltpu

_SPECIES = (1, 6, 7, 8)
_HIGH_CUTOFF = 6.0
_N_GAUSS = 20
_W = 2.0
_B, _N = 16, 96

_OFFSETS = np.linspace(0.0, _HIGH_CUTOFF, _N_GAUSS + 1, dtype=np.float32)[1:]
_SQRTPI = float(np.sqrt(np.pi))
_PI = float(np.pi)
_SQRT2 = float(np.sqrt(2.0))

_ANG_L = (
    ((((0, 0, 0), 1.0),)),
    (((1, 0, 0), 1.0), ((0, 1, 0), 1.0), ((0, 0, 1), 1.0)),
    (((2, 0, 0), 1.0), ((1, 1, 0), _SQRT2), ((0, 2, 0), 1.0),
     ((1, 0, 1), _SQRT2), ((0, 1, 1), _SQRT2), ((0, 0, 2), 1.0)),
)


_BPB = 8


def _fp_kernel(xc_ref, xr_ref, z_ref, cnt_ref, out_ref):
    for bi in range(_BPB):
        _fp_one(bi, xc_ref, xr_ref, z_ref, cnt_ref, out_ref)


def _fp_one(bi, xc_ref, xr_ref, z_ref, cnt_ref, out_ref):
    f32 = jnp.float32
    xc = xc_ref[bi]
    xr = xr_ref[bi]
    z = z_ref[bi]
    natom = cnt_ref[bi, 0, 0]

    n = _N
    ex = jnp.broadcast_to(xr[0:1, :], (n, n))
    ey = jnp.broadcast_to(xr[1:2, :], (n, n))
    ez = jnp.broadcast_to(xr[2:3, :], (n, n))
    dx = ex - ex.T
    dy = ey - ey.T
    dz = ez - ez.T

    d2 = jnp.maximum(dx * dx + dy * dy + dz * dz, 1e-12)
    dist = jnp.sqrt(d2)
    jj = jax.lax.broadcasted_iota(jnp.int32, (n, n), 0)
    ii = jax.lax.broadcasted_iota(jnp.int32, (n, n), 1)
    valid = (dist < _HIGH_CUTOFF) & (ii != jj) & (jj < natom)
    coeffs = valid.astype(f32)

    inv_d = 1.0 / dist
    inv_d2 = inv_d * inv_d
    cut = 0.5 * (jnp.cos(dist * _PI / _HIGH_CUTOFF) + 1.0)
    dd = dist * dist
    w_over = _W / dd
    onep = 1.0 + w_over
    sigma2 = jnp.log(onep)
    mu = jnp.log(dist / jnp.sqrt(onep))
    sqs = jnp.sqrt(sigma2)

    lane_i = jax.lax.broadcasted_iota(jnp.int32, (1, n), 1)
    valid_i = (lane_i < natom).astype(f32)
    ccoef = coeffs * valid_i

    cc2 = (cut * ccoef) / (_SQRTPI * sqs)
    ninv2s = -0.5 / sigma2
    q0 = ninv2s * mu * mu
    w1 = -(2.0 * ninv2s * mu + 1.0)

    rad = []
    for g in range(_N_GAUSS):
        k = float(np.log(np.float32(_OFFSETS[g])))
        arg = (k * k) * ninv2s + k * w1 + q0
        rad.append(cc2 * jnp.exp(arg))

    u2 = inv_d2 * coeffs
    u3 = u2 * inv_d
    u4 = u2 * inv_d2
    mono = {(0, 0, 0): None,
            (1, 0, 0): dx, (0, 1, 0): dy, (0, 0, 1): dz,
            (2, 0, 0): dx * dx, (1, 1, 0): dx * dy, (0, 2, 0): dy * dy,
            (1, 0, 1): dx * dz, (0, 1, 1): dy * dz, (0, 0, 2): dz * dz}
    ubyl = (u2, u3, u4)
    bf16 = jnp.bfloat16
    ang_by_l = []
    for l in range(3):
        lst = []
        for (nmk, sw) in _ANG_L[l]:
            m = mono[nmk]
            if m is None:
                lst.append(ubyl[l].astype(bf16))
            elif sw != 1.0:
                lst.append(((ubyl[l] * sw) * m).astype(bf16))
            else:
                lst.append((ubyl[l] * m).astype(bf16))
        ang_by_l.append(lst)
    radb = [r.astype(bf16) for r in rad]

    msk = [(z == s).astype(f32) for s in _SPECIES]
    zrow = jnp.zeros_like(msk[0])
    m24 = jnp.concatenate(
        [msk[0], msk[1], msk[2], msk[3], zrow, zrow, zrow, zrow,
         msk[1], msk[2], msk[3], msk[0], zrow, zrow, zrow, zrow,
         msk[2], msk[3], msk[0], msk[1], zrow, zrow, zrow, zrow],
        axis=0)

    for l in range(3):
        angs = ang_by_l[l]
        for g in range(_N_GAUSS):
            r = radb[g]
            a_sq = a_c1 = a_c2 = None
            for a_arr in angs:
                p = a_arr * r
                t24 = jax.lax.dot(m24, p, preferred_element_type=f32)
                t = t24[0:8]
                t1 = t24[8:16]
                t2 = t24[16:24]
                if a_sq is None:
                    a_sq, a_c1, a_c2 = t * t, t * t1, t * t2
                else:
                    a_sq += t * t
                    a_c1 += t * t1
                    a_c2 += t * t2
            c1 = 2.0 * a_c1
            c2 = 2.0 * a_c2
            blk = jnp.concatenate(
                [a_sq[0:4], c1[0:1], c2[0:1], c1[3:4], c1[1:2], c2[1:2],
                 c1[2:3]],
                axis=0)
            out_ref[bi, l * _N_GAUSS + g] = blk


def kernel(coordinates, nuclear_charges, natom_counts):
    b, n, _ = coordinates.shape
    xc = coordinates.astype(jnp.float32)
    xr = jnp.transpose(xc, (0, 2, 1))
    z = nuclear_charges.astype(jnp.int32).reshape(b, 1, n)
    cnt = natom_counts.astype(jnp.int32).reshape(b, 1, 1)

    out = pl.pallas_call(
        _fp_kernel,
        grid=(b // _BPB,),
        in_specs=[
            pl.BlockSpec((_BPB, n, 3), lambda i: (i, 0, 0)),
            pl.BlockSpec((_BPB, 3, n), lambda i: (i, 0, 0)),
            pl.BlockSpec((_BPB, 1, n), lambda i: (i, 0, 0)),
            pl.BlockSpec((_BPB, 1, 1), lambda i: (i, 0, 0)),
        ],
        out_specs=pl.BlockSpec((_BPB, 60, 10, n), lambda i: (i, 0, 0, 0)),
        out_shape=jax.ShapeDtypeStruct((b, 60, 10, n), jnp.float32),
        compiler_params=pltpu.CompilerParams(
            dimension_semantics=("parallel",)),
    )(xc, xr, z, cnt)

    fp = out.reshape(b, 3, _N_GAUSS, 10, n)
    fp = jnp.transpose(fp, (0, 4, 1, 3, 2))
    return fp.reshape(b, n, 3 * 10 * _N_GAUSS)


# speedup vs baseline: 1.0167x; 1.0167x over previous
import jax
import jax.numpy as jnp
import numpy as np
from jax.experimental import pallas as pl
from jax.experimental.pallas import tpu as pltpu

_SPECIES = (1, 6, 7, 8)
_HIGH_CUTOFF = 6.0
_N_GAUSS = 20
_W = 2.0
_B, _N = 16, 96

_OFFSETS = np.linspace(0.0, _HIGH_CUTOFF, _N_GAUSS + 1, dtype=np.float32)[1:]
_SQRTPI = float(np.sqrt(np.pi))
_PI = float(np.pi)
_SQRT2 = float(np.sqrt(2.0))

_ANG_L = (
    ((((0, 0, 0), 1.0),)),
    (((1, 0, 0), 1.0), ((0, 1, 0), 1.0), ((0, 0, 1), 1.0)),
    (((2, 0, 0), 1.0), ((1, 1, 0), _SQRT2), ((0, 2, 0), 1.0),
     ((1, 0, 1), _SQRT2), ((0, 1, 1), _SQRT2), ((0, 0, 2), 1.0)),
)


_BPB = 4


def _fp_kernel(xc_ref, xr_ref, z_ref, cnt_ref, out_ref):
    for bi in range(_BPB):
        _fp_one(bi, xc_ref, xr_ref, z_ref, cnt_ref, out_ref)


def _fp_one(bi, xc_ref, xr_ref, z_ref, cnt_ref, out_ref):
    f32 = jnp.float32
    xc = xc_ref[bi]
    xr = xr_ref[bi]
    z = z_ref[bi]
    natom = cnt_ref[bi, 0, 0]

    n = _N
    ex = jnp.broadcast_to(xr[0:1, :], (n, n))
    ey = jnp.broadcast_to(xr[1:2, :], (n, n))
    ez = jnp.broadcast_to(xr[2:3, :], (n, n))
    dx = ex - ex.T
    dy = ey - ey.T
    dz = ez - ez.T

    d2 = jnp.maximum(dx * dx + dy * dy + dz * dz, 1e-12)
    dist = jnp.sqrt(d2)
    jj = jax.lax.broadcasted_iota(jnp.int32, (n, n), 0)
    ii = jax.lax.broadcasted_iota(jnp.int32, (n, n), 1)
    valid = (dist < _HIGH_CUTOFF) & (ii != jj) & (jj < natom)
    coeffs = valid.astype(f32)

    inv_d = 1.0 / dist
    inv_d2 = inv_d * inv_d
    cut = 0.5 * (jnp.cos(dist * _PI / _HIGH_CUTOFF) + 1.0)
    dd = dist * dist
    w_over = _W / dd
    onep = 1.0 + w_over
    sigma2 = jnp.log(onep)
    mu = jnp.log(dist / jnp.sqrt(onep))
    sqs = jnp.sqrt(sigma2)

    lane_i = jax.lax.broadcasted_iota(jnp.int32, (1, n), 1)
    valid_i = (lane_i < natom).astype(f32)
    ccoef = coeffs * valid_i

    cc2 = (cut * ccoef) / (_SQRTPI * sqs)
    ninv2s = -0.5 / sigma2
    q0 = ninv2s * mu * mu
    w1 = -(2.0 * ninv2s * mu + 1.0)

    rad = []
    for g in range(_N_GAUSS):
        k = float(np.log(np.float32(_OFFSETS[g])))
        arg = (k * k) * ninv2s + k * w1 + q0
        rad.append(cc2 * jnp.exp(arg))

    u2 = inv_d2 * coeffs
    u3 = u2 * inv_d
    u4 = u2 * inv_d2
    mono = {(0, 0, 0): None,
            (1, 0, 0): dx, (0, 1, 0): dy, (0, 0, 1): dz,
            (2, 0, 0): dx * dx, (1, 1, 0): dx * dy, (0, 2, 0): dy * dy,
            (1, 0, 1): dx * dz, (0, 1, 1): dy * dz, (0, 0, 2): dz * dz}
    ubyl = (u2, u3, u4)
    bf16 = jnp.bfloat16
    ang_by_l = []
    for l in range(3):
        lst = []
        for (nmk, sw) in _ANG_L[l]:
            m = mono[nmk]
            if m is None:
                lst.append(ubyl[l].astype(bf16))
            elif sw != 1.0:
                lst.append(((ubyl[l] * sw) * m).astype(bf16))
            else:
                lst.append((ubyl[l] * m).astype(bf16))
        ang_by_l.append(lst)
    radb = [r.astype(bf16) for r in rad]

    msk = [(z == s).astype(f32) for s in _SPECIES]
    msk2 = [2.0 * m for m in msk]
    zrow = jnp.zeros_like(msk[0])
    m32 = jnp.concatenate(
        [msk[0], msk[1], msk[2], msk[3],
         msk[0], msk[0], msk[0], msk[1], msk[1], msk[2],
         zrow, zrow, zrow, zrow, zrow, zrow,
         msk[0], msk[1], msk[2], msk[3],
         msk2[1], msk2[2], msk2[3], msk2[2], msk2[3], msk2[3],
         zrow, zrow, zrow, zrow, zrow, zrow],
        axis=0)

    for l in range(3):
        angs = ang_by_l[l]
        for g in range(_N_GAUSS):
            r = radb[g]
            acc = None
            for a_arr in angs:
                p = a_arr * r
                t32 = jax.lax.dot(m32, p, preferred_element_type=f32)
                prod = t32[0:16] * t32[16:32]
                acc = prod if acc is None else acc + prod
            out_ref[bi, l * _N_GAUSS + g] = acc[0:10]


def kernel(coordinates, nuclear_charges, natom_counts):
    b, n, _ = coordinates.shape
    xc = coordinates.astype(jnp.float32)
    xr = jnp.transpose(xc, (0, 2, 1))
    z = nuclear_charges.astype(jnp.int32).reshape(b, 1, n)
    cnt = natom_counts.astype(jnp.int32).reshape(b, 1, 1)

    out = pl.pallas_call(
        _fp_kernel,
        grid=(b // _BPB,),
        in_specs=[
            pl.BlockSpec((_BPB, n, 3), lambda i: (i, 0, 0)),
            pl.BlockSpec((_BPB, 3, n), lambda i: (i, 0, 0)),
            pl.BlockSpec((_BPB, 1, n), lambda i: (i, 0, 0)),
            pl.BlockSpec((_BPB, 1, 1), lambda i: (i, 0, 0)),
        ],
        out_specs=pl.BlockSpec((_BPB, 60, 10, n), lambda i: (i, 0, 0, 0)),
        out_shape=jax.ShapeDtypeStruct((b, 60, 10, n), jnp.float32),
        compiler_params=pltpu.CompilerParams(
            dimension_semantics=("parallel",)),
    )(xc, xr, z, cnt)

    fp = out.reshape(b, 3, _N_GAUSS, 10, n)
    fp = jnp.transpose(fp, (0, 4, 1, 3, 2))
    return fp.reshape(b, n, 3 * 10 * _N_GAUSS)


# speedup vs baseline: 1.0180x; 1.0012x over previous
import jax
import jax.numpy as jnp
import numpy as np
from jax.experimental import pallas as pl
from jax.experimental.pallas import tpu as pltpu

_SPECIES = (1, 6, 7, 8)
_HIGH_CUTOFF = 6.0
_N_GAUSS = 20
_W = 2.0
_B, _N = 16, 96

_OFFSETS = np.linspace(0.0, _HIGH_CUTOFF, _N_GAUSS + 1, dtype=np.float32)[1:]
_SQRTPI = float(np.sqrt(np.pi))
_PI = float(np.pi)
_SQRT2 = float(np.sqrt(2.0))

_ANG_L = (
    ((((0, 0, 0), 1.0),)),
    (((1, 0, 0), 1.0), ((0, 1, 0), 1.0), ((0, 0, 1), 1.0)),
    (((2, 0, 0), 1.0), ((1, 1, 0), _SQRT2), ((0, 2, 0), 1.0),
     ((1, 0, 1), _SQRT2), ((0, 1, 1), _SQRT2), ((0, 0, 2), 1.0)),
)


_BPB = 8


def _fp_kernel(xc_ref, xr_ref, z_ref, cnt_ref, out_ref):
    for bi in range(_BPB):
        _fp_one(bi, xc_ref, xr_ref, z_ref, cnt_ref, out_ref)


def _fp_one(bi, xc_ref, xr_ref, z_ref, cnt_ref, out_ref):
    f32 = jnp.float32
    xc = xc_ref[bi]
    xr = xr_ref[bi]
    z = z_ref[bi]
    natom = cnt_ref[bi, 0, 0]

    n = _N
    ex = jnp.broadcast_to(xr[0:1, :], (n, n))
    ey = jnp.broadcast_to(xr[1:2, :], (n, n))
    ez = jnp.broadcast_to(xr[2:3, :], (n, n))
    dx = ex - ex.T
    dy = ey - ey.T
    dz = ez - ez.T

    d2 = jnp.maximum(dx * dx + dy * dy + dz * dz, 1e-12)
    dist = jnp.sqrt(d2)
    jj = jax.lax.broadcasted_iota(jnp.int32, (n, n), 0)
    ii = jax.lax.broadcasted_iota(jnp.int32, (n, n), 1)
    valid = (dist < _HIGH_CUTOFF) & (ii != jj) & (jj < natom)
    coeffs = valid.astype(f32)

    inv_d = 1.0 / dist
    inv_d2 = inv_d * inv_d
    cut = 0.5 * (jnp.cos(dist * _PI / _HIGH_CUTOFF) + 1.0)
    dd = dist * dist
    w_over = _W / dd
    onep = 1.0 + w_over
    sigma2 = jnp.log(onep)
    mu = jnp.log(dist / jnp.sqrt(onep))
    sqs = jnp.sqrt(sigma2)

    lane_i = jax.lax.broadcasted_iota(jnp.int32, (1, n), 1)
    valid_i = (lane_i < natom).astype(f32)
    ccoef = coeffs * valid_i

    cc2 = (cut * ccoef) / (_SQRTPI * sqs)
    ninv2s = -0.5 / sigma2
    q0 = ninv2s * mu * mu
    w1 = -(2.0 * ninv2s * mu + 1.0)

    rad = []
    for g in range(_N_GAUSS):
        k = float(np.log(np.float32(_OFFSETS[g])))
        arg = (k * k) * ninv2s + k * w1 + q0
        rad.append(cc2 * jnp.exp(arg))

    u2 = inv_d2 * coeffs
    u3 = u2 * inv_d
    u4 = u2 * inv_d2
    mono = {(0, 0, 0): None,
            (1, 0, 0): dx, (0, 1, 0): dy, (0, 0, 1): dz,
            (2, 0, 0): dx * dx, (1, 1, 0): dx * dy, (0, 2, 0): dy * dy,
            (1, 0, 1): dx * dz, (0, 1, 1): dy * dz, (0, 0, 2): dz * dz}
    ubyl = (u2, u3, u4)
    bf16 = jnp.bfloat16
    ang_by_l = []
    for l in range(3):
        lst = []
        for (nmk, sw) in _ANG_L[l]:
            m = mono[nmk]
            if m is None:
                lst.append(ubyl[l].astype(bf16))
            elif sw != 1.0:
                lst.append(((ubyl[l] * sw) * m).astype(bf16))
            else:
                lst.append((ubyl[l] * m).astype(bf16))
        ang_by_l.append(lst)
    radb = [r.astype(bf16) for r in rad]

    msk = [(z == s).astype(f32) for s in _SPECIES]
    msk2 = [2.0 * m for m in msk]
    zrow = jnp.zeros_like(msk[0])
    m32 = jnp.concatenate(
        [msk[0], msk[1], msk[2], msk[3],
         msk[0], msk[0], msk[0], msk[1], msk[1], msk[2],
         zrow, zrow, zrow, zrow, zrow, zrow,
         msk[0], msk[1], msk[2], msk[3],
         msk2[1], msk2[2], msk2[3], msk2[2], msk2[3], msk2[3],
         zrow, zrow, zrow, zrow, zrow, zrow],
        axis=0)

    for l in range(3):
        angs = ang_by_l[l]
        for g in range(_N_GAUSS):
            r = radb[g]
            acc = None
            for a_arr in angs:
                p = a_arr * r
                t32 = jax.lax.dot(m32, p, preferred_element_type=f32)
                prod = t32[0:16] * t32[16:32]
                acc = prod if acc is None else acc + prod
            out_ref[bi, l * _N_GAUSS + g] = acc[0:10]


def kernel(coordinates, nuclear_charges, natom_counts):
    b, n, _ = coordinates.shape
    xc = coordinates.astype(jnp.float32)
    xr = jnp.transpose(xc, (0, 2, 1))
    z = nuclear_charges.astype(jnp.int32).reshape(b, 1, n)
    cnt = natom_counts.astype(jnp.int32).reshape(b, 1, 1)

    out = pl.pallas_call(
        _fp_kernel,
        grid=(b // _BPB,),
        in_specs=[
            pl.BlockSpec((_BPB, n, 3), lambda i: (i, 0, 0)),
            pl.BlockSpec((_BPB, 3, n), lambda i: (i, 0, 0)),
            pl.BlockSpec((_BPB, 1, n), lambda i: (i, 0, 0)),
            pl.BlockSpec((_BPB, 1, 1), lambda i: (i, 0, 0)),
        ],
        out_specs=pl.BlockSpec((_BPB, 60, 10, n), lambda i: (i, 0, 0, 0)),
        out_shape=jax.ShapeDtypeStruct((b, 60, 10, n), jnp.float32),
        compiler_params=pltpu.CompilerParams(
            dimension_semantics=("parallel",)),
    )(xc, xr, z, cnt)

    fp = out.reshape(b, 3, _N_GAUSS, 10, n)
    fp = jnp.transpose(fp, (0, 4, 1, 3, 2))
    return fp.reshape(b, n, 3 * 10 * _N_GAUSS)
